# baseline (speedup 1.0000x reference)
import functools

import jax
import jax.numpy as jnp
from jax import lax
from jax.experimental import pallas as pl
from jax.experimental.pallas import tpu as pltpu
from jax.experimental.pallas import tpu_sc as plsc

B = 16384
L = 50
V = 1000000
D = 32
C = 4
T = B * L
CPAD = 16

NC = 2
NS = 16
NW = NC * NS

BAGS_PER_W = B // NW
BAGS_PER_CHUNK = 64
TOK_PER_CHUNK = BAGS_PER_CHUNK * L
CHUNKS = BAGS_PER_W // BAGS_PER_CHUNK
GATHER_SIZES = [128] * (TOK_PER_CHUNK // 128) + (
    [TOK_PER_CHUNK % 128] if TOK_PER_CHUNK % 128 else [])

_PROJ_NB = 8192


def _tc_proj_body(w_ref, t_ref, o_ref):
  o_ref[...] = lax.dot_general(
      w_ref[...], t_ref[...],
      dimension_numbers=(((1,), (0,)), ((), ())),
      preferred_element_type=jnp.float32)


def _tc_proj(wpad, tab_t):
  grid = pl.cdiv(V, _PROJ_NB)
  return pl.pallas_call(
      _tc_proj_body,
      grid=(grid,),
      in_specs=[
          pl.BlockSpec((CPAD, D), lambda i: (0, 0)),
          pl.BlockSpec((D, _PROJ_NB), lambda i: (0, i)),
      ],
      out_specs=pl.BlockSpec((CPAD, _PROJ_NB), lambda i: (0, i)),
      out_shape=jax.ShapeDtypeStruct((CPAD, V), jnp.float32),
  )(wpad, tab_t)


def _tree_sum(vals):
  while len(vals) > 1:
    nxt = [vals[i] + vals[i + 1] for i in range(0, len(vals) - 1, 2)]
    if len(vals) % 2:
      nxt.append(vals[-1])
    vals = nxt
  return vals[0]


def _make_sc_pool():
  mesh = plsc.VectorSubcoreMesh(core_axis_name="c", subcore_axis_name="s")

  @functools.partial(
      pl.kernel,
      out_type=jax.ShapeDtypeStruct((B, CPAD), jnp.float32),
      mesh=mesh,
      scratch_types=[
          pltpu.VMEM((2, TOK_PER_CHUNK), jnp.int32),
          pltpu.VMEM((TOK_PER_CHUNK, CPAD), jnp.float32),
          pltpu.VMEM((TOK_PER_CHUNK, CPAD), jnp.float32),
          pltpu.VMEM((BAGS_PER_CHUNK, CPAD), jnp.float32),
          pltpu.VMEM((CPAD,), jnp.float32),
          pltpu.SemaphoreType.DMA,
          pltpu.SemaphoreType.DMA,
      ],
      compiler_params=pltpu.CompilerParams(use_tc_tiling_on_sc=False),
  )
  def sc_pool(text_hbm, proj_hbm, bias_hbm, out_hbm, idx_v, rows_a, rows_b,
              acc_v, bias_v, sem_a, sem_b):
    wid = lax.axis_index("s") * NC + lax.axis_index("c")
    tok_base = wid * (BAGS_PER_W * L)
    rows_p = (rows_a, rows_b)
    sem_p = (sem_a, sem_b)
    pltpu.sync_copy(bias_hbm, bias_v)

    def fire(ch, p):
      pltpu.sync_copy(
          text_hbm.at[pl.ds(tok_base + ch * TOK_PER_CHUNK, TOK_PER_CHUNK)],
          idx_v.at[p])
      off = 0
      for g in GATHER_SIZES:
        pltpu.make_async_copy(
            proj_hbm.at[idx_v.at[p, pl.ds(off, g)]],
            rows_p[p].at[pl.ds(off, g)], sem_p[p]).start()
        off += g

    def drain(p):
      off = 0
      for g in GATHER_SIZES:
        pltpu.make_async_copy(
            proj_hbm.at[idx_v.at[p, pl.ds(off, g)]],
            rows_p[p].at[pl.ds(off, g)], sem_p[p]).wait()
        off += g

    def compute(ch, p):
      rows_v = rows_p[p]
      bias = bias_v[...]

      def bag_body(i, carry2):
        base = i * L
        acc_v[i, :] = _tree_sum(
            [rows_v[base + t, :] for t in range(L)]) + bias
        return carry2

      lax.fori_loop(0, BAGS_PER_CHUNK, bag_body, 0)
      pltpu.sync_copy(
          acc_v,
          out_hbm.at[pl.ds(wid * BAGS_PER_W + ch * BAGS_PER_CHUNK,
                           BAGS_PER_CHUNK)])

    fire(0, 0)
    fire(1, 1)

    def superstep(ss, carry):
      for p in range(2):
        ch = ss * 2 + p
        drain(p)
        compute(ch, p)

        @pl.when(ch < CHUNKS - 2)
        def _():
          fire(ch + 2, p)
      return carry

    lax.fori_loop(0, CHUNKS // 2, superstep, 0)

  return sc_pool


_sc_pool = _make_sc_pool()


def kernel(text, offsets, table, W, b):
  del offsets
  wpad = jnp.zeros((CPAD, D), jnp.float32).at[:C].set(W / jnp.float32(L))
  bpad = jnp.zeros((CPAD,), jnp.float32).at[:C].set(b)
  proj_t = _tc_proj(wpad, table.T)
  out16 = _sc_pool(text, proj_t.T, bpad)
  return out16[:, :C]


# speedup vs baseline: 1.2025x; 1.2025x over previous
import functools

import jax
import jax.numpy as jnp
from jax import lax
from jax.experimental import pallas as pl
from jax.experimental.pallas import tpu as pltpu
from jax.experimental.pallas import tpu_sc as plsc

B = 16384
L = 50
V = 1000000
D = 32
C = 4
T = B * L
CPAD = 16

NC = 2
NS = 16
NW = NC * NS

BAGS_PER_W = B // NW
BAGS_PER_CHUNK = 64
TOK_PER_CHUNK = BAGS_PER_CHUNK * L
CHUNKS = BAGS_PER_W // BAGS_PER_CHUNK
GATHER_SIZES = [128] * (TOK_PER_CHUNK // 128) + (
    [TOK_PER_CHUNK % 128] if TOK_PER_CHUNK % 128 else [])

_PROJ_NB = 8192


def _tc_proj_body(w_ref, t_ref, o_ref):
  o_ref[...] = lax.dot_general(
      t_ref[...], w_ref[...],
      dimension_numbers=(((0,), (1,)), ((), ())),
      preferred_element_type=jnp.float32)


def _tc_proj(wpad, tab_t):
  grid = pl.cdiv(V, _PROJ_NB)
  return pl.pallas_call(
      _tc_proj_body,
      grid=(grid,),
      in_specs=[
          pl.BlockSpec((CPAD, D), lambda i: (0, 0)),
          pl.BlockSpec((D, _PROJ_NB), lambda i: (0, i)),
      ],
      out_specs=pl.BlockSpec((_PROJ_NB, CPAD), lambda i: (i, 0)),
      out_shape=jax.ShapeDtypeStruct((V, CPAD), jnp.float32),
  )(wpad, tab_t)


def _tree_sum(vals):
  while len(vals) > 1:
    nxt = [vals[i] + vals[i + 1] for i in range(0, len(vals) - 1, 2)]
    if len(vals) % 2:
      nxt.append(vals[-1])
    vals = nxt
  return vals[0]


def _make_sc_pool():
  mesh = plsc.VectorSubcoreMesh(core_axis_name="c", subcore_axis_name="s")

  @functools.partial(
      pl.kernel,
      out_type=jax.ShapeDtypeStruct((B, CPAD), jnp.float32),
      mesh=mesh,
      scratch_types=[
          pltpu.VMEM((2, TOK_PER_CHUNK), jnp.int32),
          pltpu.VMEM((TOK_PER_CHUNK, CPAD), jnp.float32),
          pltpu.VMEM((TOK_PER_CHUNK, CPAD), jnp.float32),
          pltpu.VMEM((BAGS_PER_CHUNK, CPAD), jnp.float32),
          pltpu.VMEM((CPAD,), jnp.float32),
          pltpu.SemaphoreType.DMA,
          pltpu.SemaphoreType.DMA,
      ],
      compiler_params=pltpu.CompilerParams(use_tc_tiling_on_sc=False),
  )
  def sc_pool(text_hbm, proj_hbm, bias_hbm, out_hbm, idx_v, rows_a, rows_b,
              acc_v, bias_v, sem_a, sem_b):
    wid = lax.axis_index("s") * NC + lax.axis_index("c")
    tok_base = wid * (BAGS_PER_W * L)
    rows_p = (rows_a, rows_b)
    sem_p = (sem_a, sem_b)
    pltpu.sync_copy(bias_hbm, bias_v)

    def fire(ch, p):
      pltpu.sync_copy(
          text_hbm.at[pl.ds(tok_base + ch * TOK_PER_CHUNK, TOK_PER_CHUNK)],
          idx_v.at[p])
      off = 0
      for g in GATHER_SIZES:
        pltpu.make_async_copy(
            proj_hbm.at[idx_v.at[p, pl.ds(off, g)]],
            rows_p[p].at[pl.ds(off, g)], sem_p[p]).start()
        off += g

    def drain(p):
      off = 0
      for g in GATHER_SIZES:
        pltpu.make_async_copy(
            proj_hbm.at[idx_v.at[p, pl.ds(off, g)]],
            rows_p[p].at[pl.ds(off, g)], sem_p[p]).wait()
        off += g

    def compute(ch, p):
      rows_v = rows_p[p]
      bias = bias_v[...]

      def bag_body(i, carry2):
        base = i * L
        acc_v[i, :] = _tree_sum(
            [rows_v[base + t, :] for t in range(L)]) + bias
        return carry2

      lax.fori_loop(0, BAGS_PER_CHUNK, bag_body, 0)
      pltpu.sync_copy(
          acc_v,
          out_hbm.at[pl.ds(wid * BAGS_PER_W + ch * BAGS_PER_CHUNK,
                           BAGS_PER_CHUNK)])

    fire(0, 0)
    fire(1, 1)

    def superstep(ss, carry):
      for p in range(2):
        ch = ss * 2 + p
        drain(p)
        compute(ch, p)

        @pl.when(ch < CHUNKS - 2)
        def _():
          fire(ch + 2, p)
      return carry

    lax.fori_loop(0, CHUNKS // 2, superstep, 0)

  return sc_pool


_sc_pool = _make_sc_pool()


def kernel(text, offsets, table, W, b):
  del offsets
  wpad = jnp.zeros((CPAD, D), jnp.float32).at[:C].set(W / jnp.float32(L))
  bpad = jnp.zeros((CPAD,), jnp.float32).at[:C].set(b)
  proj_v = _tc_proj(wpad, table.T)
  out16 = _sc_pool(text, proj_v, bpad)
  return out16[:, :C]


# speedup vs baseline: 2.1729x; 1.8070x over previous
import functools

import jax
import jax.numpy as jnp
from jax import lax
from jax.experimental import pallas as pl
from jax.experimental.pallas import tpu as pltpu
from jax.experimental.pallas import tpu_sc as plsc

B = 16384
L = 50
V = 1000000
D = 32
C = 4
T = B * L
CPAD = 16

NC = 2
NS = 16
NW = NC * NS

BAGS_PER_W = B // NW
BAGS_PER_CHUNK = 64
TOK_PER_CHUNK = BAGS_PER_CHUNK * L
CHUNKS = BAGS_PER_W // BAGS_PER_CHUNK
GATHER_SIZES = [128] * (TOK_PER_CHUNK // 128) + (
    [TOK_PER_CHUNK % 128] if TOK_PER_CHUNK % 128 else [])

_PROJ_NB = 8192


_PROJ_GRID = pl.cdiv(V, _PROJ_NB)
VP = _PROJ_GRID * _PROJ_NB
_PROJ_SUB = _PROJ_NB // 8


def _tc_proj_body(w_ref, t_ref, o_ref):
  p = lax.dot_general(
      t_ref[...], w_ref[...],
      dimension_numbers=(((0,), (1,)), ((), ())),
      preferred_element_type=jnp.float32)
  o_ref[...] = jnp.concatenate(
      [p[_PROJ_SUB * j:_PROJ_SUB * (j + 1), :] for j in range(8)], axis=1)


def _tc_proj(wpad, tab_t):
  return pl.pallas_call(
      _tc_proj_body,
      grid=(_PROJ_GRID,),
      in_specs=[
          pl.BlockSpec((CPAD, D), lambda i: (0, 0)),
          pl.BlockSpec((D, _PROJ_NB), lambda i: (0, i)),
      ],
      out_specs=pl.BlockSpec((_PROJ_SUB, 128), lambda i: (i, 0)),
      out_shape=jax.ShapeDtypeStruct((VP // 8, 128), jnp.float32),
  )(wpad, tab_t)


def _tree_sum(vals):
  while len(vals) > 1:
    nxt = [vals[i] + vals[i + 1] for i in range(0, len(vals) - 1, 2)]
    if len(vals) % 2:
      nxt.append(vals[-1])
    vals = nxt
  return vals[0]


def _make_sc_pool():
  mesh = plsc.VectorSubcoreMesh(core_axis_name="c", subcore_axis_name="s")

  @functools.partial(
      pl.kernel,
      out_type=jax.ShapeDtypeStruct((B, CPAD), jnp.float32),
      mesh=mesh,
      scratch_types=[
          pltpu.VMEM((2, TOK_PER_CHUNK), jnp.int32),
          pltpu.VMEM((TOK_PER_CHUNK, CPAD), jnp.float32),
          pltpu.VMEM((TOK_PER_CHUNK, CPAD), jnp.float32),
          pltpu.VMEM((BAGS_PER_CHUNK, CPAD), jnp.float32),
          pltpu.VMEM((CPAD,), jnp.float32),
          pltpu.SemaphoreType.DMA,
          pltpu.SemaphoreType.DMA,
      ],
      compiler_params=pltpu.CompilerParams(use_tc_tiling_on_sc=False),
  )
  def sc_pool(text_hbm, proj_hbm, bias_hbm, out_hbm, idx_v, rows_a, rows_b,
              acc_v, bias_v, sem_a, sem_b):
    wid = lax.axis_index("s") * NC + lax.axis_index("c")
    tok_base = wid * (BAGS_PER_W * L)
    rows_p = (rows_a, rows_b)
    sem_p = (sem_a, sem_b)
    pltpu.sync_copy(bias_hbm, bias_v)

    def fire(ch, p):
      pltpu.sync_copy(
          text_hbm.at[pl.ds(tok_base + ch * TOK_PER_CHUNK, TOK_PER_CHUNK)],
          idx_v.at[p])

      def remap_body(i, carry):
        t = idx_v[p, pl.ds(i * 16, 16)]
        k = ((t & jnp.int32(-8192)) | ((t & jnp.int32(1023)) << 3)
             | ((t >> 10) & jnp.int32(7)))
        idx_v[p, pl.ds(i * 16, 16)] = k
        return carry

      lax.fori_loop(0, TOK_PER_CHUNK // 16, remap_body, 0)
      off = 0
      for g in GATHER_SIZES:
        pltpu.make_async_copy(
            proj_hbm.at[idx_v.at[p, pl.ds(off, g)]],
            rows_p[p].at[pl.ds(off, g)], sem_p[p]).start()
        off += g

    def drain(p):
      off = 0
      for g in GATHER_SIZES:
        pltpu.make_async_copy(
            proj_hbm.at[idx_v.at[p, pl.ds(off, g)]],
            rows_p[p].at[pl.ds(off, g)], sem_p[p]).wait()
        off += g

    def compute(ch, p):
      rows_v = rows_p[p]
      bias = bias_v[...]

      def bag_body(i, carry2):
        base = i * L
        acc_v[i, :] = _tree_sum(
            [rows_v[base + t, :] for t in range(L)]) + bias
        return carry2

      lax.fori_loop(0, BAGS_PER_CHUNK, bag_body, 0)
      pltpu.sync_copy(
          acc_v,
          out_hbm.at[pl.ds(wid * BAGS_PER_W + ch * BAGS_PER_CHUNK,
                           BAGS_PER_CHUNK)])

    fire(0, 0)
    fire(1, 1)

    def superstep(ss, carry):
      for p in range(2):
        ch = ss * 2 + p
        drain(p)
        compute(ch, p)

        @pl.when(ch < CHUNKS - 2)
        def _():
          fire(ch + 2, p)
      return carry

    lax.fori_loop(0, CHUNKS // 2, superstep, 0)

  return sc_pool


_sc_pool = _make_sc_pool()


def kernel(text, offsets, table, W, b):
  del offsets
  wpad = jnp.zeros((CPAD, D), jnp.float32).at[:C].set(W / jnp.float32(L))
  bpad = jnp.zeros((CPAD,), jnp.float32).at[:C].set(b)
  proj128 = _tc_proj(wpad, table.T)
  proj_v = jnp.reshape(proj128, (VP, CPAD))
  out16 = _sc_pool(text, proj_v, bpad)
  return out16[:, :C]


# speedup vs baseline: 3.3725x; 1.5521x over previous
import functools

import jax
import jax.numpy as jnp
from jax import lax
from jax.experimental import pallas as pl
from jax.experimental.pallas import tpu as pltpu
from jax.experimental.pallas import tpu_sc as plsc

B = 16384
L = 50
V = 1000000
D = 32
C = 4
T = B * L
CPAD = 16

NC = 2
NS = 16
NW = NC * NS

BAGS_PER_W = B // NW
BAGS_PER_CHUNK = 64
TOK_PER_CHUNK = BAGS_PER_CHUNK * L
CHUNKS = BAGS_PER_W // BAGS_PER_CHUNK
GATHER_SIZES = [128] * (TOK_PER_CHUNK // 128) + (
    [TOK_PER_CHUNK % 128] if TOK_PER_CHUNK % 128 else [])

_PROJ_NB = 8192


_PROJ_GRID = pl.cdiv(V, _PROJ_NB)
VP = _PROJ_GRID * _PROJ_NB
_PROJ_SUB = _PROJ_NB // 8


def _tc_proj_body(f_ref, t_ref, o_ref):
  dots = [
      lax.dot_general(
          t_ref[:, pl.ds(_PROJ_SUB * j, _PROJ_SUB)].astype(jnp.bfloat16),
          f_ref[j].astype(jnp.bfloat16),
          dimension_numbers=(((0,), (0,)), ((), ())),
          preferred_element_type=jnp.float32)
      for j in range(8)
  ]
  o_ref[...] = _tree_sum(dots)


def _tc_proj(fmat, tab_t):
  return pl.pallas_call(
      _tc_proj_body,
      grid=(_PROJ_GRID,),
      in_specs=[
          pl.BlockSpec((8, D, 128), lambda i: (0, 0, 0)),
          pl.BlockSpec((D, _PROJ_NB), lambda i: (0, i)),
      ],
      out_specs=pl.BlockSpec((_PROJ_SUB, 128), lambda i: (i, 0)),
      out_shape=jax.ShapeDtypeStruct((VP // 8, 128), jnp.float32),
  )(fmat, tab_t)


def _tree_sum(vals):
  while len(vals) > 1:
    nxt = [vals[i] + vals[i + 1] for i in range(0, len(vals) - 1, 2)]
    if len(vals) % 2:
      nxt.append(vals[-1])
    vals = nxt
  return vals[0]


def _make_sc_pool():
  mesh = plsc.VectorSubcoreMesh(core_axis_name="c", subcore_axis_name="s")

  @functools.partial(
      pl.kernel,
      out_type=jax.ShapeDtypeStruct((B, CPAD), jnp.float32),
      mesh=mesh,
      scratch_types=[
          pltpu.VMEM((2, TOK_PER_CHUNK), jnp.int32),
          pltpu.VMEM((TOK_PER_CHUNK, CPAD), jnp.float32),
          pltpu.VMEM((TOK_PER_CHUNK, CPAD), jnp.float32),
          pltpu.VMEM((BAGS_PER_CHUNK, CPAD), jnp.float32),
          pltpu.VMEM((CPAD,), jnp.float32),
          pltpu.SemaphoreType.DMA,
          pltpu.SemaphoreType.DMA,
      ],
      compiler_params=pltpu.CompilerParams(use_tc_tiling_on_sc=False),
  )
  def sc_pool(text_hbm, proj_hbm, bias_hbm, out_hbm, idx_v, rows_a, rows_b,
              acc_v, bias_v, sem_a, sem_b):
    wid = lax.axis_index("s") * NC + lax.axis_index("c")
    tok_base = wid * (BAGS_PER_W * L)
    rows_p = (rows_a, rows_b)
    sem_p = (sem_a, sem_b)
    pltpu.sync_copy(bias_hbm, bias_v)

    def fire(ch, p):
      pltpu.sync_copy(
          text_hbm.at[pl.ds(tok_base + ch * TOK_PER_CHUNK, TOK_PER_CHUNK)],
          idx_v.at[p])

      def remap_body(i, carry):
        t = idx_v[p, pl.ds(i * 16, 16)]
        k = ((t & jnp.int32(-8192)) | ((t & jnp.int32(1023)) << 3)
             | ((t >> 10) & jnp.int32(7)))
        idx_v[p, pl.ds(i * 16, 16)] = k
        return carry

      lax.fori_loop(0, TOK_PER_CHUNK // 16, remap_body, 0)
      off = 0
      for g in GATHER_SIZES:
        pltpu.make_async_copy(
            proj_hbm.at[idx_v.at[p, pl.ds(off, g)]],
            rows_p[p].at[pl.ds(off, g)], sem_p[p]).start()
        off += g

    def drain(p):
      off = 0
      for g in GATHER_SIZES:
        pltpu.make_async_copy(
            proj_hbm.at[idx_v.at[p, pl.ds(off, g)]],
            rows_p[p].at[pl.ds(off, g)], sem_p[p]).wait()
        off += g

    def compute(ch, p):
      rows_v = rows_p[p]
      bias = bias_v[...]

      def bag_body(i, carry2):
        base = i * L
        acc_v[i, :] = _tree_sum(
            [rows_v[base + t, :] for t in range(L)]) + bias
        return carry2

      lax.fori_loop(0, BAGS_PER_CHUNK, bag_body, 0)
      pltpu.sync_copy(
          acc_v,
          out_hbm.at[pl.ds(wid * BAGS_PER_W + ch * BAGS_PER_CHUNK,
                           BAGS_PER_CHUNK)])

    fire(0, 0)
    fire(1, 1)

    def superstep(ss, carry):
      for p in range(2):
        ch = ss * 2 + p
        drain(p)
        compute(ch, p)

        @pl.when(ch < CHUNKS - 2)
        def _():
          fire(ch + 2, p)
      return carry

    lax.fori_loop(0, CHUNKS // 2, superstep, 0)

  return sc_pool


_sc_pool = _make_sc_pool()


def kernel(text, offsets, table, W, b):
  del offsets
  wpad = jnp.zeros((CPAD, D), jnp.float32).at[:C].set(W / jnp.float32(L))
  bpad = jnp.zeros((CPAD,), jnp.float32).at[:C].set(b)
  fmat = jnp.zeros((8, D, 128), jnp.float32)
  for j in range(8):
    fmat = fmat.at[j, :, CPAD * j:CPAD * (j + 1)].set(wpad.T)
  proj128 = _tc_proj(fmat, table.T)
  proj_v = jnp.reshape(proj128, (VP, CPAD))
  out16 = _sc_pool(text, proj_v, bpad)
  return out16[:, :C]


# speedup vs baseline: 4.6634x; 1.3828x over previous
import functools

import jax
import jax.numpy as jnp
from jax import lax
from jax.experimental import pallas as pl
from jax.experimental.pallas import tpu as pltpu
from jax.experimental.pallas import tpu_sc as plsc

B = 16384
L = 50
V = 1000000
D = 32
C = 4
T = B * L
CPAD = 16

NC = 2
NS = 16
NW = NC * NS

BAGS_PER_W = B // NW
BAGS_PER_CHUNK = 64
TOK_PER_CHUNK = BAGS_PER_CHUNK * L
CHUNKS = BAGS_PER_W // BAGS_PER_CHUNK
GATHER_SIZES = [128] * (TOK_PER_CHUNK // 128) + (
    [TOK_PER_CHUNK % 128] if TOK_PER_CHUNK % 128 else [])

_PROJ_NB = 65536


_PROJ_GRID = pl.cdiv(V, _PROJ_NB)
VP = _PROJ_GRID * _PROJ_NB
_PROJ_SUB = _PROJ_NB // 8


def _tc_proj_body(f_ref, t_ref, o_ref):
  dots = [
      lax.dot_general(
          t_ref[:, pl.ds(_PROJ_SUB * j, _PROJ_SUB)].astype(jnp.bfloat16),
          f_ref[j].astype(jnp.bfloat16),
          dimension_numbers=(((0,), (0,)), ((), ())),
          preferred_element_type=jnp.float32)
      for j in range(8)
  ]
  o_ref[...] = _tree_sum(dots)


def _tc_proj(fmat, tab_t):
  return pl.pallas_call(
      _tc_proj_body,
      grid=(_PROJ_GRID,),
      in_specs=[
          pl.BlockSpec((8, D, 128), lambda i: (0, 0, 0)),
          pl.BlockSpec((D, _PROJ_NB), lambda i: (0, i)),
      ],
      out_specs=pl.BlockSpec((_PROJ_SUB, 128), lambda i: (i, 0)),
      out_shape=jax.ShapeDtypeStruct((VP // 8, 128), jnp.float32),
  )(fmat, tab_t)


def _tree_sum(vals):
  while len(vals) > 1:
    nxt = [vals[i] + vals[i + 1] for i in range(0, len(vals) - 1, 2)]
    if len(vals) % 2:
      nxt.append(vals[-1])
    vals = nxt
  return vals[0]


def _make_sc_pool():
  mesh = plsc.VectorSubcoreMesh(core_axis_name="c", subcore_axis_name="s")

  @functools.partial(
      pl.kernel,
      out_type=jax.ShapeDtypeStruct((B, CPAD), jnp.float32),
      mesh=mesh,
      scratch_types=[
          pltpu.VMEM((2, TOK_PER_CHUNK), jnp.int32),
          pltpu.VMEM((TOK_PER_CHUNK, CPAD), jnp.float32),
          pltpu.VMEM((TOK_PER_CHUNK, CPAD), jnp.float32),
          pltpu.VMEM((BAGS_PER_CHUNK, CPAD), jnp.float32),
          pltpu.VMEM((CPAD,), jnp.float32),
          pltpu.SemaphoreType.DMA,
          pltpu.SemaphoreType.DMA,
      ],
      compiler_params=pltpu.CompilerParams(use_tc_tiling_on_sc=False),
  )
  def sc_pool(text_hbm, proj_hbm, bias_hbm, out_hbm, idx_v, rows_a, rows_b,
              acc_v, bias_v, sem_a, sem_b):
    wid = lax.axis_index("s") * NC + lax.axis_index("c")
    tok_base = wid * (BAGS_PER_W * L)
    rows_p = (rows_a, rows_b)
    sem_p = (sem_a, sem_b)
    pltpu.sync_copy(bias_hbm, bias_v)

    def fire(ch, p):
      pltpu.sync_copy(
          text_hbm.at[pl.ds(tok_base + ch * TOK_PER_CHUNK, TOK_PER_CHUNK)],
          idx_v.at[p])

      sub_shift = _PROJ_SUB.bit_length() - 1

      def remap_body(i, carry):
        t = idx_v[p, pl.ds(i * 16, 16)]
        k = ((t & jnp.int32(-_PROJ_NB))
             | ((t & jnp.int32(_PROJ_SUB - 1)) << 3)
             | ((t >> sub_shift) & jnp.int32(7)))
        idx_v[p, pl.ds(i * 16, 16)] = k
        return carry

      lax.fori_loop(0, TOK_PER_CHUNK // 16, remap_body, 0)
      off = 0
      for g in GATHER_SIZES:
        pltpu.make_async_copy(
            proj_hbm.at[idx_v.at[p, pl.ds(off, g)]],
            rows_p[p].at[pl.ds(off, g)], sem_p[p]).start()
        off += g

    def drain(p):
      off = 0
      for g in GATHER_SIZES:
        pltpu.make_async_copy(
            proj_hbm.at[idx_v.at[p, pl.ds(off, g)]],
            rows_p[p].at[pl.ds(off, g)], sem_p[p]).wait()
        off += g

    def compute(ch, p):
      rows_v = rows_p[p]
      bias = bias_v[...]

      def bag_body(i, carry2):
        base = i * L
        acc_v[i, :] = _tree_sum(
            [rows_v[base + t, :] for t in range(L)]) + bias
        return carry2

      lax.fori_loop(0, BAGS_PER_CHUNK, bag_body, 0)
      pltpu.sync_copy(
          acc_v,
          out_hbm.at[pl.ds(wid * BAGS_PER_W + ch * BAGS_PER_CHUNK,
                           BAGS_PER_CHUNK)])

    fire(0, 0)
    fire(1, 1)

    def superstep(ss, carry):
      for p in range(2):
        ch = ss * 2 + p
        drain(p)
        compute(ch, p)

        @pl.when(ch < CHUNKS - 2)
        def _():
          fire(ch + 2, p)
      return carry

    lax.fori_loop(0, CHUNKS // 2, superstep, 0)

  return sc_pool


_sc_pool = _make_sc_pool()


def kernel(text, offsets, table, W, b):
  del offsets
  wpad = jnp.zeros((CPAD, D), jnp.float32).at[:C].set(W / jnp.float32(L))
  bpad = jnp.zeros((CPAD,), jnp.float32).at[:C].set(b)
  fmat = jnp.zeros((8, D, 128), jnp.float32)
  for j in range(8):
    fmat = fmat.at[j, :, CPAD * j:CPAD * (j + 1)].set(wpad.T)
  proj128 = _tc_proj(fmat, table.T)
  proj_v = jnp.reshape(proj128, (VP, CPAD))
  out16 = _sc_pool(text, proj_v, bpad)
  return out16[:, :C]


# speedup vs baseline: 4.7904x; 1.0272x over previous
import functools

import jax
import jax.numpy as jnp
from jax import lax
from jax.experimental import pallas as pl
from jax.experimental.pallas import tpu as pltpu
from jax.experimental.pallas import tpu_sc as plsc

B = 16384
L = 50
V = 1000000
D = 32
C = 4
T = B * L
CPAD = 16

NC = 2
NS = 16
NW = NC * NS

BAGS_PER_W = B // NW
BAGS_PER_CHUNK = 64
TOK_PER_CHUNK = BAGS_PER_CHUNK * L
CHUNKS = BAGS_PER_W // BAGS_PER_CHUNK
GATHER_SIZES = [128] * (TOK_PER_CHUNK // 128) + (
    [TOK_PER_CHUNK % 128] if TOK_PER_CHUNK % 128 else [])

_PROJ_NB = 65536


_PROJ_GRID = pl.cdiv(V, _PROJ_NB)
VP = _PROJ_GRID * _PROJ_NB
_PROJ_SUB = _PROJ_NB // 8


def _tc_proj_body(f_ref, t_ref, o_ref):
  dots = [
      lax.dot_general(
          t_ref[:, pl.ds(_PROJ_SUB * j, _PROJ_SUB)].astype(jnp.bfloat16),
          f_ref[j].astype(jnp.bfloat16),
          dimension_numbers=(((0,), (0,)), ((), ())),
          preferred_element_type=jnp.float32)
      for j in range(8)
  ]
  o_ref[...] = _tree_sum(dots)


def _tc_proj(fmat, tab_t):
  return pl.pallas_call(
      _tc_proj_body,
      grid=(_PROJ_GRID,),
      in_specs=[
          pl.BlockSpec((8, D, 128), lambda i: (0, 0, 0)),
          pl.BlockSpec((D, _PROJ_NB), lambda i: (0, i)),
      ],
      out_specs=pl.BlockSpec((_PROJ_SUB, 128), lambda i: (i, 0)),
      out_shape=jax.ShapeDtypeStruct((VP // 8, 128), jnp.float32),
  )(fmat, tab_t)


def _tree_sum(vals):
  while len(vals) > 1:
    nxt = [vals[i] + vals[i + 1] for i in range(0, len(vals) - 1, 2)]
    if len(vals) % 2:
      nxt.append(vals[-1])
    vals = nxt
  return vals[0]


def _make_sc_pool():
  mesh = plsc.VectorSubcoreMesh(core_axis_name="c", subcore_axis_name="s")

  @functools.partial(
      pl.kernel,
      out_type=jax.ShapeDtypeStruct((B, CPAD), jnp.float32),
      mesh=mesh,
      scratch_types=[
          pltpu.VMEM((2, TOK_PER_CHUNK), jnp.int32),
          pltpu.VMEM((TOK_PER_CHUNK, CPAD), jnp.float32),
          pltpu.VMEM((TOK_PER_CHUNK, CPAD), jnp.float32),
          pltpu.VMEM((BAGS_PER_CHUNK, CPAD), jnp.float32),
          pltpu.VMEM((CPAD,), jnp.float32),
          pltpu.SemaphoreType.DMA,
          pltpu.SemaphoreType.DMA,
      ],
      compiler_params=pltpu.CompilerParams(use_tc_tiling_on_sc=False),
  )
  def sc_pool(text_hbm, proj_hbm, bias_hbm, out_hbm, idx_v, rows_a, rows_b,
              acc_v, bias_v, sem_a, sem_b):
    wid = lax.axis_index("s") * NC + lax.axis_index("c")
    tok_base = wid * (BAGS_PER_W * L)
    rows_p = (rows_a, rows_b)
    sem_p = (sem_a, sem_b)
    pltpu.sync_copy(bias_hbm, bias_v)

    def fire(ch, p):
      pltpu.sync_copy(
          text_hbm.at[pl.ds(tok_base + ch * TOK_PER_CHUNK, TOK_PER_CHUNK)],
          idx_v.at[p])

      sub_shift = _PROJ_SUB.bit_length() - 1

      def remap_body(i, carry):
        for u in range(4):
          s = i * 64 + u * 16
          t = idx_v[p, pl.ds(s, 16)]
          k = ((t & jnp.int32(-_PROJ_NB))
               | ((t & jnp.int32(_PROJ_SUB - 1)) << 3)
               | ((t >> sub_shift) & jnp.int32(7)))
          idx_v[p, pl.ds(s, 16)] = k
        return carry

      lax.fori_loop(0, TOK_PER_CHUNK // 64, remap_body, 0)
      off = 0
      for g in GATHER_SIZES:
        pltpu.make_async_copy(
            proj_hbm.at[idx_v.at[p, pl.ds(off, g)]],
            rows_p[p].at[pl.ds(off, g)], sem_p[p]).start()
        off += g

    def drain(p):
      off = 0
      for g in GATHER_SIZES:
        pltpu.make_async_copy(
            proj_hbm.at[idx_v.at[p, pl.ds(off, g)]],
            rows_p[p].at[pl.ds(off, g)], sem_p[p]).wait()
        off += g

    def compute(ch, p):
      rows_v = rows_p[p]
      bias = bias_v[...]

      def bag_body(i, carry2):
        base = i * L
        acc_v[i, :] = _tree_sum(
            [rows_v[base + t, :] for t in range(L)]) + bias
        return carry2

      lax.fori_loop(0, BAGS_PER_CHUNK, bag_body, 0)
      pltpu.sync_copy(
          acc_v,
          out_hbm.at[pl.ds(wid * BAGS_PER_W + ch * BAGS_PER_CHUNK,
                           BAGS_PER_CHUNK)])

    fire(0, 0)
    fire(1, 1)

    def superstep(ss, carry):
      for p in range(2):
        ch = ss * 2 + p
        drain(p)
        compute(ch, p)

        @pl.when(ch < CHUNKS - 2)
        def _():
          fire(ch + 2, p)
      return carry

    lax.fori_loop(0, CHUNKS // 2, superstep, 0)

  return sc_pool


_sc_pool = _make_sc_pool()


def kernel(text, offsets, table, W, b):
  del offsets
  wpad = jnp.zeros((CPAD, D), jnp.float32).at[:C].set(W / jnp.float32(L))
  bpad = jnp.zeros((CPAD,), jnp.float32).at[:C].set(b)
  fmat = jnp.zeros((8, D, 128), jnp.float32)
  for j in range(8):
    fmat = fmat.at[j, :, CPAD * j:CPAD * (j + 1)].set(wpad.T)
  proj128 = _tc_proj(fmat, table.T)
  proj_v = jnp.reshape(proj128, (VP, CPAD))
  out16 = _sc_pool(text, proj_v, bpad)
  return out16[:, :C]


# speedup vs baseline: 5.3520x; 1.1172x over previous
import functools

import jax
import jax.numpy as jnp
from jax import lax
from jax.experimental import pallas as pl
from jax.experimental.pallas import tpu as pltpu
from jax.experimental.pallas import tpu_sc as plsc

B = 16384
L = 50
V = 1000000
D = 32
C = 4
T = B * L
CPAD = 16

NC = 2
NS = 16
NW = NC * NS

BAGS_PER_W = B // NW
BAGS_PER_CHUNK = 64
TOK_PER_CHUNK = BAGS_PER_CHUNK * L
CHUNKS = BAGS_PER_W // BAGS_PER_CHUNK
GATHER_SIZES = [128] * (TOK_PER_CHUNK // 128) + (
    [TOK_PER_CHUNK % 128] if TOK_PER_CHUNK % 128 else [])

_PROJ_NB = 65536


_PROJ_GRID = pl.cdiv(V, _PROJ_NB)
VP = _PROJ_GRID * _PROJ_NB
_PROJ_SUB = _PROJ_NB // 8


def _tc_proj_body(f_ref, t_ref, o_ref):
  lhs = jnp.concatenate(
      [t_ref[:, pl.ds(_PROJ_SUB * j, _PROJ_SUB)] for j in range(8)],
      axis=0).astype(jnp.bfloat16)
  o_ref[...] = lax.dot_general(
      lhs, f_ref[...].astype(jnp.bfloat16),
      dimension_numbers=(((0,), (0,)), ((), ())),
      preferred_element_type=jnp.float32)


def _tc_proj(fmat, tab_t):
  return pl.pallas_call(
      _tc_proj_body,
      grid=(_PROJ_GRID,),
      in_specs=[
          pl.BlockSpec((8 * D, 128), lambda i: (0, 0)),
          pl.BlockSpec((D, _PROJ_NB), lambda i: (0, i)),
      ],
      out_specs=pl.BlockSpec((_PROJ_SUB, 128), lambda i: (i, 0)),
      out_shape=jax.ShapeDtypeStruct((VP // 8, 128), jnp.float32),
  )(fmat, tab_t)


def _tree_sum(vals):
  while len(vals) > 1:
    nxt = [vals[i] + vals[i + 1] for i in range(0, len(vals) - 1, 2)]
    if len(vals) % 2:
      nxt.append(vals[-1])
    vals = nxt
  return vals[0]


def _make_sc_pool():
  mesh = plsc.VectorSubcoreMesh(core_axis_name="c", subcore_axis_name="s")

  @functools.partial(
      pl.kernel,
      out_type=jax.ShapeDtypeStruct((B, CPAD), jnp.float32),
      mesh=mesh,
      scratch_types=[
          pltpu.VMEM((2, TOK_PER_CHUNK), jnp.int32),
          pltpu.VMEM((TOK_PER_CHUNK, CPAD), jnp.float32),
          pltpu.VMEM((TOK_PER_CHUNK, CPAD), jnp.float32),
          pltpu.VMEM((BAGS_PER_CHUNK, CPAD), jnp.float32),
          pltpu.VMEM((CPAD,), jnp.float32),
          pltpu.SemaphoreType.DMA,
          pltpu.SemaphoreType.DMA,
      ],
      compiler_params=pltpu.CompilerParams(use_tc_tiling_on_sc=False),
  )
  def sc_pool(text_hbm, proj_hbm, bias_hbm, out_hbm, idx_v, rows_a, rows_b,
              acc_v, bias_v, sem_a, sem_b):
    wid = lax.axis_index("s") * NC + lax.axis_index("c")
    tok_base = wid * (BAGS_PER_W * L)
    rows_p = (rows_a, rows_b)
    sem_p = (sem_a, sem_b)
    pltpu.sync_copy(bias_hbm, bias_v)

    def fire(ch, p):
      pltpu.sync_copy(
          text_hbm.at[pl.ds(tok_base + ch * TOK_PER_CHUNK, TOK_PER_CHUNK)],
          idx_v.at[p])

      sub_shift = _PROJ_SUB.bit_length() - 1

      def remap_body(i, carry):
        for u in range(4):
          s = i * 64 + u * 16
          t = idx_v[p, pl.ds(s, 16)]
          k = ((t & jnp.int32(-_PROJ_NB))
               | ((t & jnp.int32(_PROJ_SUB - 1)) << 3)
               | ((t >> sub_shift) & jnp.int32(7)))
          idx_v[p, pl.ds(s, 16)] = k
        return carry

      lax.fori_loop(0, TOK_PER_CHUNK // 64, remap_body, 0)
      off = 0
      for g in GATHER_SIZES:
        pltpu.make_async_copy(
            proj_hbm.at[idx_v.at[p, pl.ds(off, g)]],
            rows_p[p].at[pl.ds(off, g)], sem_p[p]).start()
        off += g

    def drain(p):
      off = 0
      for g in GATHER_SIZES:
        pltpu.make_async_copy(
            proj_hbm.at[idx_v.at[p, pl.ds(off, g)]],
            rows_p[p].at[pl.ds(off, g)], sem_p[p]).wait()
        off += g

    def compute(ch, p):
      rows_v = rows_p[p]
      bias = bias_v[...]

      def bag_body(i, carry2):
        base = i * L
        acc_v[i, :] = _tree_sum(
            [rows_v[base + t, :] for t in range(L)]) + bias
        return carry2

      lax.fori_loop(0, BAGS_PER_CHUNK, bag_body, 0)
      pltpu.sync_copy(
          acc_v,
          out_hbm.at[pl.ds(wid * BAGS_PER_W + ch * BAGS_PER_CHUNK,
                           BAGS_PER_CHUNK)])

    fire(0, 0)
    fire(1, 1)

    def superstep(ss, carry):
      for p in range(2):
        ch = ss * 2 + p
        drain(p)
        compute(ch, p)

        @pl.when(ch < CHUNKS - 2)
        def _():
          fire(ch + 2, p)
      return carry

    lax.fori_loop(0, CHUNKS // 2, superstep, 0)

  return sc_pool


_sc_pool = _make_sc_pool()


def kernel(text, offsets, table, W, b):
  del offsets
  wpad = jnp.zeros((CPAD, D), jnp.float32).at[:C].set(W / jnp.float32(L))
  bpad = jnp.zeros((CPAD,), jnp.float32).at[:C].set(b)
  fmat = jnp.zeros((8 * D, 128), jnp.float32)
  for j in range(8):
    fmat = fmat.at[D * j:D * (j + 1), CPAD * j:CPAD * (j + 1)].set(wpad.T)
  proj128 = _tc_proj(fmat, table.T)
  proj_v = jnp.reshape(proj128, (VP, CPAD))
  out16 = _sc_pool(text, proj_v, bpad)
  return out16[:, :C]
